# Initial kernel scaffold; baseline (speedup 1.0000x reference)
import jax
import jax.numpy as jnp
from jax.experimental import pallas as pl


def kernel(position_ids, embeddings):
    raise NotImplementedError("write your pallas kernel here")


# speedup vs baseline: 2.3187x; 2.3187x over previous
import math
import functools

import jax
import jax.numpy as jnp
from jax.experimental import pallas as pl

_DIM = 1024
_NEG_LOG_FREQ_OVER_DIM = -math.log(10000.0) / _DIM


def _pe_add_block(emb_ref, out_ref, *, s_blk):
    base = pl.program_id(0) * s_blk
    row = jax.lax.broadcasted_iota(jnp.int32, (s_blk, _DIM), 0).astype(jnp.float32)
    lane = jax.lax.broadcasted_iota(jnp.int32, (s_blk, _DIM), 1)
    even_lane = lane - (lane % 2)
    inv_freq = jnp.exp(even_lane.astype(jnp.float32) * _NEG_LOG_FREQ_OVER_DIM)
    ang = (row + base) * inv_freq
    pe = jnp.where(lane % 2 == 0, jnp.sin(ang), jnp.cos(ang))
    out_ref[...] = emb_ref[...] + pe[None]


@jax.jit
def kernel(position_ids, embeddings):
    batch, max_len, dim = embeddings.shape
    s_blk = 256
    grid = (max_len // s_blk,)
    return pl.pallas_call(
        functools.partial(_pe_add_block, s_blk=s_blk),
        grid=grid,
        in_specs=[
            pl.BlockSpec((batch, s_blk, dim), lambda i: (0, i, 0)),
        ],
        out_specs=pl.BlockSpec((batch, s_blk, dim), lambda i: (0, i, 0)),
        out_shape=jax.ShapeDtypeStruct(embeddings.shape, embeddings.dtype),
    )(embeddings)
